# baseline (speedup 1.0000x reference)
import functools

import jax
import jax.numpy as jnp
from jax import lax
from jax.experimental import pallas as pl
from jax.experimental.pallas import tpu as pltpu
from jax.experimental.pallas import tpu_sc as plsc

B, S = 4096, 200
VOCAB = 30000
D = 768
K1 = 1.2
BB = 0.75
EPS = 1e-10

NC, NS, L = 2, 16, 16
NW = NC * NS
DPW = B // NW
GROUP = 16
CHUNKS = ((0, 56), (56, 56), (112, 56), (168, 32))
CMAX = 56


RBLK = 256
WBLK = 32


def _weights_body(ids_ref, mask_ref, w_ref):
    ids = ids_ref[...]
    msk = mask_ref[...]
    valid = (msk == 1) & (ids > 100) & (ids < VOCAB)
    vf = valid.astype(jnp.float32)
    doc_len = jnp.sum(vf, axis=1, keepdims=True)
    denom = jnp.maximum(1.0 + BB * (doc_len / 100.0 - 1.0), 0.5)
    eq = ids[:, :, None] == ids[:, None, :]
    c = jnp.sum(jnp.where(eq, vf[:, None, :], 0.0), axis=2)
    w_ref[...] = jnp.where(valid, (K1 + 1.0) / (c + K1 * denom), 0.0)


def _weights(input_ids, attention_mask):
    return pl.pallas_call(
        _weights_body,
        grid=(B // WBLK,),
        in_specs=[
            pl.BlockSpec((WBLK, S), lambda i: (i, 0)),
            pl.BlockSpec((WBLK, S), lambda i: (i, 0)),
        ],
        out_specs=pl.BlockSpec((WBLK, S), lambda i: (i, 0)),
        out_shape=jax.ShapeDtypeStruct((B, S), jnp.float32),
    )(input_ids, attention_mask)


CS_BLK = 3000


def _colsum_body(wt_ref, u_ref):
    @pl.when(pl.program_id(0) == 0)
    def _():
        u_ref[...] = jnp.zeros_like(u_ref)

    u_ref[...] += jnp.sum(wt_ref[...], axis=0, keepdims=True)


def _colsum(wt):
    return pl.pallas_call(
        _colsum_body,
        grid=(VOCAB // CS_BLK,),
        in_specs=[pl.BlockSpec((CS_BLK, D), lambda i: (i, 0))],
        out_specs=pl.BlockSpec((1, D), lambda i: (0, 0)),
        out_shape=jax.ShapeDtypeStruct((1, D), jnp.float32),
    )(wt)


def _sc_body(wt_hbm, ids_hbm, w_hbm, out_hbm,
             ids_v, w_v, rows0, rows1, acc, sem0, sem1):
    wid = lax.axis_index("s") * NC + lax.axis_index("c")
    base = wid * DPW
    bufs = (rows0, rows1)
    sems = (sem0, sem1)

    def group_body(g, _):
        g0 = base + g * GROUP
        tok0 = pl.multiple_of(g0 * S, 8)
        pltpu.sync_copy(ids_hbm.at[pl.ds(tok0, GROUP * S)], ids_v)
        pltpu.sync_copy(w_hbm.at[pl.ds(tok0, GROUP * S)], w_v)

        def doc_body(d, _):
            def zero_body(i, _):
                acc[pl.ds(i * L, L)] = jnp.zeros((L,), jnp.float32)
                return 0

            lax.fori_loop(0, D // L, zero_body, 0)

            def start(ci):
                off, sz = CHUNKS[ci]
                return pltpu.async_copy(
                    wt_hbm.at[ids_v.at[pl.ds(pl.multiple_of(d * S + off, 8), sz)]],
                    bufs[ci % 2].at[pl.ds(0, sz)],
                    sems[ci % 2],
                )

            descs = [None] * len(CHUNKS)
            descs[0] = start(0)
            for ci, (off, sz) in enumerate(CHUNKS):
                if ci + 1 < len(CHUNKS):
                    descs[ci + 1] = start(ci + 1)
                descs[ci].wait()
                buf = bufs[ci % 2]

                def row_body(r, _):
                    wb = plsc.load_gather(
                        w_v, [jnp.full((L,), d * S + off, jnp.int32) + r])

                    def vec_body(i, _):
                        prod = buf[r, pl.ds(i * L, L)] * wb
                        plsc.addupdate(acc.at[pl.ds(i * L, L)], prod)
                        return 0

                    lax.fori_loop(0, D // L, vec_body, 0)
                    return 0

                lax.fori_loop(0, sz, row_body, 0)

            pltpu.sync_copy(
                acc, out_hbm.at[pl.ds(pl.multiple_of((g0 + d) * D, 8), D)])
            return 0

        lax.fori_loop(0, GROUP, doc_body, 0)
        return 0

    lax.fori_loop(0, DPW // GROUP, group_body, 0)


def _sc_accumulate(wt, input_ids, w):
    mesh = plsc.VectorSubcoreMesh(
        core_axis_name="c", subcore_axis_name="s",
        num_cores=NC, num_subcores=NS,
    )
    f = functools.partial(
        pl.kernel,
        out_type=jax.ShapeDtypeStruct((B * D,), jnp.float32),
        mesh=mesh,
        compiler_params=pltpu.CompilerParams(needs_layout_passes=False),
        scratch_types=[
            pltpu.VMEM((GROUP * S,), jnp.int32),
            pltpu.VMEM((GROUP * S,), jnp.float32),
            pltpu.VMEM((CMAX, D), jnp.float32),
            pltpu.VMEM((CMAX, D), jnp.float32),
            pltpu.VMEM((D,), jnp.float32),
            pltpu.SemaphoreType.DMA,
            pltpu.SemaphoreType.DMA,
        ],
    )(_sc_body)
    return f(wt, input_ids.reshape(-1), w.reshape(-1)).reshape(B, D)


def _norm_body(a_ref, u_ref, o_ref):
    t = a_ref[...] + EPS * u_ref[...]
    ss = jnp.sum(t * t, axis=1, keepdims=True)
    n = jnp.maximum(jnp.sqrt(ss), 1e-12)
    o_ref[...] = t / n


def _normalize(acc, u):
    return pl.pallas_call(
        _norm_body,
        grid=(B // RBLK,),
        in_specs=[
            pl.BlockSpec((RBLK, D), lambda i: (i, 0)),
            pl.BlockSpec((1, D), lambda i: (0, 0)),
        ],
        out_specs=pl.BlockSpec((RBLK, D), lambda i: (i, 0)),
        out_shape=jax.ShapeDtypeStruct((B, D), jnp.float32),
    )(acc, u)


def kernel(input_ids, attention_mask, W):
    wt = jnp.asarray(W.T)
    w = _weights(input_ids.astype(jnp.int32), attention_mask.astype(jnp.int32))
    u = _colsum(wt)
    acc = _sc_accumulate(wt, input_ids.astype(jnp.int32), w)
    return _normalize(acc, u)


# speedup vs baseline: 1.0996x; 1.0996x over previous
import functools

import jax
import jax.numpy as jnp
from jax import lax
from jax.experimental import pallas as pl
from jax.experimental.pallas import tpu as pltpu
from jax.experimental.pallas import tpu_sc as plsc

B, S = 4096, 200
VOCAB = 30000
D = 768
K1 = 1.2
BB = 0.75
EPS = 1e-10

NC, NS, L = 2, 16, 16
NW = NC * NS
DPW = B // NW
GROUP = 16
CHUNKS = ((0, 56), (56, 56), (112, 56), (168, 32))
CMAX = 56


RBLK = 256
WBLK = 32


def _weights_body(ids_ref, mask_ref, w_ref):
    ids = ids_ref[...]
    msk = mask_ref[...]
    valid = (msk == 1) & (ids > 100) & (ids < VOCAB)
    vf = valid.astype(jnp.float32)
    doc_len = jnp.sum(vf, axis=1, keepdims=True)
    denom = jnp.maximum(1.0 + BB * (doc_len / 100.0 - 1.0), 0.5)
    eq = ids[:, :, None] == ids[:, None, :]
    c = jnp.sum(jnp.where(eq, vf[:, None, :], 0.0), axis=2)
    w_ref[...] = jnp.where(valid, (K1 + 1.0) / (c + K1 * denom), 0.0)


def _weights(input_ids, attention_mask):
    return pl.pallas_call(
        _weights_body,
        grid=(B // WBLK,),
        in_specs=[
            pl.BlockSpec((WBLK, S), lambda i: (i, 0)),
            pl.BlockSpec((WBLK, S), lambda i: (i, 0)),
        ],
        out_specs=pl.BlockSpec((WBLK, S), lambda i: (i, 0)),
        out_shape=jax.ShapeDtypeStruct((B, S), jnp.float32),
    )(input_ids, attention_mask)


CS_BLK = 3000


def _colsum_body(wt_ref, u_ref):
    @pl.when(pl.program_id(0) == 0)
    def _():
        u_ref[...] = jnp.zeros_like(u_ref)

    u_ref[...] += jnp.sum(wt_ref[...], axis=0, keepdims=True)


def _colsum(wt):
    return pl.pallas_call(
        _colsum_body,
        grid=(VOCAB // CS_BLK,),
        in_specs=[pl.BlockSpec((CS_BLK, D), lambda i: (i, 0))],
        out_specs=pl.BlockSpec((1, D), lambda i: (0, 0)),
        out_shape=jax.ShapeDtypeStruct((1, D), jnp.float32),
    )(wt)


def _sc_body(wt_hbm, ids_hbm, w_hbm, out_hbm,
             ids_v, w_v, rows0, rows1, acc, sem0, sem1):
    wid = lax.axis_index("s") * NC + lax.axis_index("c")
    base = wid * DPW
    bufs = (rows0, rows1)
    sems = (sem0, sem1)

    def group_body(g, _):
        g0 = base + g * GROUP
        tok0 = pl.multiple_of(g0 * S, 8)
        pltpu.sync_copy(ids_hbm.at[pl.ds(tok0, GROUP * S)], ids_v)
        pltpu.sync_copy(w_hbm.at[pl.ds(tok0, GROUP * S)], w_v)

        def doc_body(d, _):
            for i in range(D // L):
                acc[pl.ds(i * L, L)] = jnp.zeros((L,), jnp.float32)

            def start(ci):
                off, sz = CHUNKS[ci]
                return pltpu.async_copy(
                    wt_hbm.at[ids_v.at[pl.ds(pl.multiple_of(d * S + off, 8), sz)]],
                    bufs[ci % 2].at[pl.ds(0, sz)],
                    sems[ci % 2],
                )

            descs = [None] * len(CHUNKS)
            descs[0] = start(0)
            for ci, (off, sz) in enumerate(CHUNKS):
                if ci + 1 < len(CHUNKS):
                    descs[ci + 1] = start(ci + 1)
                descs[ci].wait()
                buf = bufs[ci % 2]

                def row_body(r, _):
                    wb = plsc.load_gather(
                        w_v, [jnp.full((L,), d * S + off, jnp.int32) + r])
                    for i in range(D // L):
                        prod = buf[r, pl.ds(i * L, L)] * wb
                        plsc.addupdate(acc.at[pl.ds(i * L, L)], prod)
                    return 0

                lax.fori_loop(0, sz, row_body, 0)

            pltpu.sync_copy(
                acc, out_hbm.at[pl.ds(pl.multiple_of((g0 + d) * D, 8), D)])
            return 0

        lax.fori_loop(0, GROUP, doc_body, 0)
        return 0

    lax.fori_loop(0, DPW // GROUP, group_body, 0)


def _sc_accumulate(wt, input_ids, w):
    mesh = plsc.VectorSubcoreMesh(
        core_axis_name="c", subcore_axis_name="s",
        num_cores=NC, num_subcores=NS,
    )
    f = functools.partial(
        pl.kernel,
        out_type=jax.ShapeDtypeStruct((B * D,), jnp.float32),
        mesh=mesh,
        compiler_params=pltpu.CompilerParams(needs_layout_passes=False),
        scratch_types=[
            pltpu.VMEM((GROUP * S,), jnp.int32),
            pltpu.VMEM((GROUP * S,), jnp.float32),
            pltpu.VMEM((CMAX, D), jnp.float32),
            pltpu.VMEM((CMAX, D), jnp.float32),
            pltpu.VMEM((D,), jnp.float32),
            pltpu.SemaphoreType.DMA,
            pltpu.SemaphoreType.DMA,
        ],
    )(_sc_body)
    return f(wt, input_ids.reshape(-1), w.reshape(-1)).reshape(B, D)


def _norm_body(a_ref, u_ref, o_ref):
    t = a_ref[...] + EPS * u_ref[...]
    ss = jnp.sum(t * t, axis=1, keepdims=True)
    n = jnp.maximum(jnp.sqrt(ss), 1e-12)
    o_ref[...] = t / n


def _normalize(acc, u):
    return pl.pallas_call(
        _norm_body,
        grid=(B // RBLK,),
        in_specs=[
            pl.BlockSpec((RBLK, D), lambda i: (i, 0)),
            pl.BlockSpec((1, D), lambda i: (0, 0)),
        ],
        out_specs=pl.BlockSpec((RBLK, D), lambda i: (i, 0)),
        out_shape=jax.ShapeDtypeStruct((B, D), jnp.float32),
    )(acc, u)


def kernel(input_ids, attention_mask, W):
    wt = jnp.asarray(W.T)
    w = _weights(input_ids.astype(jnp.int32), attention_mask.astype(jnp.int32))
    u = _colsum(wt)
    acc = _sc_accumulate(wt, input_ids.astype(jnp.int32), w)
    return _normalize(acc, u)


# speedup vs baseline: 4.0277x; 3.6628x over previous
import functools

import jax
import jax.numpy as jnp
from jax import lax
from jax.experimental import pallas as pl
from jax.experimental.pallas import tpu as pltpu
from jax.experimental.pallas import tpu_sc as plsc

B, S = 4096, 200
VOCAB = 30000
D = 768
K1 = 1.2
BB = 0.75
EPS = 1e-10

NC, NS, L = 2, 16, 16
NW = NC * NS
DPW = B // NW
GROUP = 16
CHUNKS = ((0, 56), (56, 56), (112, 56), (168, 32))
CMAX = 56


RBLK = 256
WBLK = 32


def _weights_body(ids_ref, mask_ref, w_ref):
    ids = ids_ref[...]
    msk = mask_ref[...]
    valid = (msk == 1) & (ids > 100) & (ids < VOCAB)
    vf = valid.astype(jnp.float32)
    doc_len = jnp.sum(vf, axis=1, keepdims=True)
    denom = jnp.maximum(1.0 + BB * (doc_len / 100.0 - 1.0), 0.5)
    eq = ids[:, :, None] == ids[:, None, :]
    c = jnp.sum(jnp.where(eq, vf[:, None, :], 0.0), axis=2)
    w_ref[...] = jnp.where(valid, (K1 + 1.0) / (c + K1 * denom), 0.0)


def _weights(input_ids, attention_mask):
    return pl.pallas_call(
        _weights_body,
        grid=(B // WBLK,),
        in_specs=[
            pl.BlockSpec((WBLK, S), lambda i: (i, 0)),
            pl.BlockSpec((WBLK, S), lambda i: (i, 0)),
        ],
        out_specs=pl.BlockSpec((WBLK, S), lambda i: (i, 0)),
        out_shape=jax.ShapeDtypeStruct((B, S), jnp.float32),
    )(input_ids, attention_mask)


CS_BLK = 3000


def _colsum_body(wt_ref, u_ref):
    @pl.when(pl.program_id(0) == 0)
    def _():
        u_ref[...] = jnp.zeros_like(u_ref)

    u_ref[...] += jnp.sum(wt_ref[...], axis=0, keepdims=True)


def _colsum(wt):
    return pl.pallas_call(
        _colsum_body,
        grid=(VOCAB // CS_BLK,),
        in_specs=[pl.BlockSpec((CS_BLK, D), lambda i: (i, 0))],
        out_specs=pl.BlockSpec((1, D), lambda i: (0, 0)),
        out_shape=jax.ShapeDtypeStruct((1, D), jnp.float32),
    )(wt)


def _sc_body(wt_hbm, ids_hbm, w_hbm, out_hbm,
             ids_v, w_v, rows0, rows1, acc, sem0, sem1):
    wid = lax.axis_index("s") * NC + lax.axis_index("c")
    base = wid * DPW
    bufs = (rows0, rows1)
    sems = (sem0, sem1)

    def group_body(g, _):
        g0 = base + g * GROUP
        tok0 = pl.multiple_of(g0 * S, 8)
        pltpu.sync_copy(ids_hbm.at[pl.ds(tok0, GROUP * S)], ids_v)
        pltpu.sync_copy(w_hbm.at[pl.ds(tok0, GROUP * S)], w_v)

        def doc_body(d, _):
            for i in range(D // L):
                acc[pl.ds(i * L, L)] = jnp.zeros((L,), jnp.float32)

            def start(ci):
                off, sz = CHUNKS[ci]
                return pltpu.async_copy(
                    wt_hbm.at[ids_v.at[pl.ds(pl.multiple_of(d * S + off, 8), sz)]],
                    bufs[ci % 2].at[pl.ds(0, sz)],
                    sems[ci % 2],
                )

            descs = [None] * len(CHUNKS)
            descs[0] = start(0)
            for ci, (off, sz) in enumerate(CHUNKS):
                if ci + 1 < len(CHUNKS):
                    descs[ci + 1] = start(ci + 1)
                descs[ci].wait()
                buf = bufs[ci % 2]

                half = D // L // 2
                for h in range(2):
                    def row_body(r, accs):
                        wb = plsc.load_gather(
                            w_v, [jnp.full((L,), d * S + off, jnp.int32) + r])
                        return tuple(
                            accs[i] + buf[r, pl.ds((h * half + i) * L, L)] * wb
                            for i in range(half))

                    zero = jnp.zeros((L,), jnp.float32)
                    accs = lax.fori_loop(0, sz, row_body, (zero,) * half)
                    for i in range(half):
                        plsc.addupdate(
                            acc.at[pl.ds((h * half + i) * L, L)], accs[i])

            pltpu.sync_copy(
                acc, out_hbm.at[pl.ds(pl.multiple_of((g0 + d) * D, 8), D)])
            return 0

        lax.fori_loop(0, GROUP, doc_body, 0)
        return 0

    lax.fori_loop(0, DPW // GROUP, group_body, 0)


def _sc_accumulate(wt, input_ids, w):
    mesh = plsc.VectorSubcoreMesh(
        core_axis_name="c", subcore_axis_name="s",
        num_cores=NC, num_subcores=NS,
    )
    f = functools.partial(
        pl.kernel,
        out_type=jax.ShapeDtypeStruct((B * D,), jnp.float32),
        mesh=mesh,
        compiler_params=pltpu.CompilerParams(needs_layout_passes=False),
        scratch_types=[
            pltpu.VMEM((GROUP * S,), jnp.int32),
            pltpu.VMEM((GROUP * S,), jnp.float32),
            pltpu.VMEM((CMAX, D), jnp.float32),
            pltpu.VMEM((CMAX, D), jnp.float32),
            pltpu.VMEM((D,), jnp.float32),
            pltpu.SemaphoreType.DMA,
            pltpu.SemaphoreType.DMA,
        ],
    )(_sc_body)
    return f(wt, input_ids.reshape(-1), w.reshape(-1)).reshape(B, D)


def _norm_body(a_ref, u_ref, o_ref):
    t = a_ref[...] + EPS * u_ref[...]
    ss = jnp.sum(t * t, axis=1, keepdims=True)
    n = jnp.maximum(jnp.sqrt(ss), 1e-12)
    o_ref[...] = t / n


def _normalize(acc, u):
    return pl.pallas_call(
        _norm_body,
        grid=(B // RBLK,),
        in_specs=[
            pl.BlockSpec((RBLK, D), lambda i: (i, 0)),
            pl.BlockSpec((1, D), lambda i: (0, 0)),
        ],
        out_specs=pl.BlockSpec((RBLK, D), lambda i: (i, 0)),
        out_shape=jax.ShapeDtypeStruct((B, D), jnp.float32),
    )(acc, u)


def kernel(input_ids, attention_mask, W):
    wt = jnp.asarray(W.T)
    w = _weights(input_ids.astype(jnp.int32), attention_mask.astype(jnp.int32))
    u = _colsum(wt)
    acc = _sc_accumulate(wt, input_ids.astype(jnp.int32), w)
    return _normalize(acc, u)
